# baseline (speedup 1.0000x reference)
import jax
import jax.numpy as jnp
from jax.experimental import pallas as pl
from jax.experimental.pallas import tpu as pltpu

_ROWS = 3 * 1080
_COLS = 1920
_CHUNKS = (128, 256, 512, 768, 768, 768, 40)
_MAXC = max(_CHUNKS)
_NBUF = 4
_OFFS = tuple(sum(_CHUNKS[:i]) for i in range(len(_CHUNKS)))
_N = len(_CHUNKS)


def _body(idx_ref, a_ref, b_ref, x_ref, o_ref, xbuf, obuf, in_sem, out_sem):
    i0 = idx_ref[0]
    scale = jnp.exp(a_ref[i0])
    shift = b_ref[i0]

    def in_copy(i):
        rows = _CHUNKS[i]
        return pltpu.make_async_copy(
            x_ref.at[pl.ds(_OFFS[i], rows)],
            xbuf.at[i % _NBUF, pl.ds(0, rows)],
            in_sem.at[i % _NBUF],
        )

    def out_copy(i):
        rows = _CHUNKS[i]
        return pltpu.make_async_copy(
            obuf.at[i % _NBUF, pl.ds(0, rows)],
            o_ref.at[pl.ds(_OFFS[i], rows)],
            out_sem.at[i % _NBUF],
        )

    for k in range(min(_NBUF, _N)):
        in_copy(k).start()
    for i in range(_N):
        in_copy(i).wait()
        if i >= _NBUF:
            out_copy(i - _NBUF).wait()
        rows = _CHUNKS[i]
        for r in range(0, rows, 128):
            sub = min(128, rows - r)
            obuf[i % _NBUF, pl.ds(r, sub), :] = (
                xbuf[i % _NBUF, pl.ds(r, sub), :] * scale + shift
            )
        out_copy(i).start()
        nxt = i + _NBUF
        if nxt < _N:
            in_copy(nxt).start()
    for i in range(max(0, _N - _NBUF), _N):
        out_copy(i).wait()


def kernel(rendered_image, cur_index, exposure_a, exposure_b):
    x2d = rendered_image.reshape(_ROWS, _COLS)
    out = pl.pallas_call(
        _body,
        in_specs=[
            pl.BlockSpec(memory_space=pltpu.SMEM),
            pl.BlockSpec(memory_space=pltpu.SMEM),
            pl.BlockSpec(memory_space=pltpu.SMEM),
            pl.BlockSpec(memory_space=pl.ANY),
        ],
        out_specs=pl.BlockSpec(memory_space=pl.ANY),
        out_shape=jax.ShapeDtypeStruct((_ROWS, _COLS), jnp.float32),
        scratch_shapes=[
            pltpu.VMEM((_NBUF, _MAXC, _COLS), jnp.float32),
            pltpu.VMEM((_NBUF, _MAXC, _COLS), jnp.float32),
            pltpu.SemaphoreType.DMA((_NBUF,)),
            pltpu.SemaphoreType.DMA((_NBUF,)),
        ],
        compiler_params=pltpu.CompilerParams(vmem_limit_bytes=100 * 1024 * 1024),
    )(cur_index, exposure_a.reshape(-1), exposure_b.reshape(-1), x2d)
    return out.reshape(rendered_image.shape)


# speedup vs baseline: 1.1660x; 1.1660x over previous
import jax
import jax.numpy as jnp
from jax.experimental import pallas as pl
from jax.experimental.pallas import tpu as pltpu

_ROWS = 3 * 1080
_COLS = 1920
_BM = 1288
_SUB = 184


def _body(idx_ref, a_ref, b_ref, x_ref, o_ref):
    i = idx_ref[0]
    scale = jnp.exp(a_ref[i])
    shift = b_ref[i]
    for r in range(0, _BM, _SUB):
        o_ref[pl.ds(r, _SUB), :] = x_ref[pl.ds(r, _SUB), :] * scale + shift


def kernel(rendered_image, cur_index, exposure_a, exposure_b):
    x2d = rendered_image.reshape(_ROWS, _COLS)
    out = pl.pallas_call(
        _body,
        grid=(pl.cdiv(_ROWS, _BM),),
        in_specs=[
            pl.BlockSpec(memory_space=pltpu.SMEM),
            pl.BlockSpec(memory_space=pltpu.SMEM),
            pl.BlockSpec(memory_space=pltpu.SMEM),
            pl.BlockSpec((_BM, _COLS), lambda i: (i, 0)),
        ],
        out_specs=pl.BlockSpec((_BM, _COLS), lambda i: (i, 0)),
        out_shape=jax.ShapeDtypeStruct((_ROWS, _COLS), jnp.float32),
        compiler_params=pltpu.CompilerParams(vmem_limit_bytes=100 * 1024 * 1024),
    )(cur_index, exposure_a.reshape(-1), exposure_b.reshape(-1), x2d)
    return out.reshape(rendered_image.shape)
